# baseline (speedup 1.0000x reference)
import functools

import jax
import jax.numpy as jnp
from jax.experimental import pallas as pl
from jax.experimental.pallas import tpu as pltpu

_CHUNK = 3072


def _reduce_kernel(x_ref, s_ref, sq_ref, n_images: int):
    step = pl.program_id(0)
    base = step * x_ref.shape[2]
    valid = n_images - base
    lane = jax.lax.broadcasted_iota(jnp.int32, x_ref.shape, 2)
    x = jnp.where(lane < valid, x_ref[...], 0.0)
    p = jnp.sum(x, axis=0)
    q = jnp.sum(x * x, axis=0)
    ps = jnp.zeros((x_ref.shape[1], 128), jnp.float32)
    qs = jnp.zeros((x_ref.shape[1], 128), jnp.float32)
    for t in range(x_ref.shape[2] // 128):
        ps = ps + p[:, t * 128:(t + 1) * 128]
        qs = qs + q[:, t * 128:(t + 1) * 128]
    s_ref[0] = ps
    sq_ref[0] = qs


def kernel(x_train, y_train, indices):
    n, h, w = x_train.shape
    xt = x_train.transpose(1, 2, 0)
    grid = (n + _CHUNK - 1) // _CHUNK

    s, sq = pl.pallas_call(
        functools.partial(_reduce_kernel, n_images=n),
        grid=(grid,),
        in_specs=[pl.BlockSpec((h, w, _CHUNK), lambda i: (0, 0, i))],
        out_specs=[
            pl.BlockSpec((1, w, 128), lambda i: (i, 0, 0)),
            pl.BlockSpec((1, w, 128), lambda i: (i, 0, 0)),
        ],
        out_shape=[
            jax.ShapeDtypeStruct((grid, w, 128), jnp.float32),
            jax.ShapeDtypeStruct((grid, w, 128), jnp.float32),
        ],
        compiler_params=pltpu.CompilerParams(
            dimension_semantics=("parallel",)),
    )(xt)
    total = jnp.float32(n * h * w)
    mean = jnp.sum(s) / total
    var = jnp.sum(sq) / total - mean * mean
    inv_std = jax.lax.rsqrt(var)
    xs = (jnp.take(x_train, indices, axis=0) - mean) * inv_std
    return xs, jnp.take(y_train, indices).sum()
